# baseline (speedup 1.0000x reference)
import jax
import jax.numpy as jnp
import numpy as np
from jax.experimental import pallas as pl
from jax.experimental.pallas import tpu as pltpu

_ROWS = 32
_V = 1000000
_SUB = 8
_LANES = _V // _SUB
_CW = 1024
_NCH = _LANES // _CW
_TAIL0 = _NCH * _CW
_TAILW = _LANES - _TAIL0

_KS0 = np.uint32(0)
_KS1 = np.uint32(1)
_KS2 = np.uint32(0x1BD11BDB)
_ROT_A = (13, 15, 26, 6)
_ROT_B = (17, 29, 16, 24)
_BIG = np.int32(2 ** 30)


def _threefry_q(j):
    x0 = jnp.zeros_like(j) + _KS0
    x1 = j + _KS1
    sched = ((_ROT_A, _KS1, _KS2, 1), (_ROT_B, _KS2, _KS0, 2),
             (_ROT_A, _KS0, _KS1, 3), (_ROT_B, _KS1, _KS2, 4),
             (_ROT_A, _KS2, _KS0, 5))
    for rots, ka, kb, c in sched:
        for r in rots:
            x0 = x0 + x1
            x1 = (x1 << np.uint32(r)) | (x1 >> np.uint32(32 - r))
            x1 = x0 ^ x1
        x0 = x0 + ka
        x1 = x1 + kb + np.uint32(c)
    bits = x0 ^ x1
    fb = (bits >> np.uint32(9)) | np.uint32(0x3F800000)
    u = jnp.maximum(np.float32(0),
                    jax.lax.bitcast_convert_type(fb, jnp.float32)
                    - np.float32(1))
    return -jnp.log1p(-u)


def _iota2(shape):
    return (jax.lax.broadcasted_iota(jnp.int32, shape, 0),
            jax.lax.broadcasted_iota(jnp.int32, shape, 1))


def _body(x_ref, p_ref, s_ref, e_ref, q_ref):
    row = pl.program_id(0)
    m = jnp.max(x_ref[0])

    sub_i, lane_i = _iota2((_SUB, _CW))
    jbase = (sub_i * _LANES + lane_i + row * _V).astype(jnp.uint32)

    def pass_b(c, s_acc):
        off = c * _CW
        x_c = x_ref[0, :, pl.ds(off, _CW)]
        q_c = _threefry_q(jbase + off.astype(jnp.uint32))
        e_c = jnp.exp(x_c - m)
        e_ref[:, pl.ds(off, _CW)] = e_c
        q_ref[:, pl.ds(off, _CW)] = q_c
        return s_acc + jnp.sum(e_c)

    s = jax.lax.fori_loop(0, _NCH, pass_b, jnp.float32(0))

    sub_t, lane_t = _iota2((_SUB, _TAILW))
    jtail = (sub_t * _LANES + lane_t + (row * _V + _TAIL0)).astype(jnp.uint32)
    x_t = x_ref[0, :, pl.ds(_TAIL0, _TAILW)]
    q_t = _threefry_q(jtail)
    e_t = jnp.exp(x_t - m)
    s = s + jnp.sum(e_t)
    p_t = e_t / s
    p_ref[0, :, pl.ds(_TAIL0, _TAILW)] = p_t
    t_t = p_t / q_t
    mt_t = jnp.max(t_t)
    flat_t = sub_t * _LANES + _TAIL0 + lane_t
    idx_t = jnp.min(jnp.where(t_t == mt_t, flat_t, _BIG))

    def pass_c(c, carry):
        best_v, best_c = carry
        off = c * _CW
        e_c = e_ref[:, pl.ds(off, _CW)]
        q_c = q_ref[:, pl.ds(off, _CW)]
        p_c = e_c / s
        p_ref[0, :, pl.ds(off, _CW)] = p_c
        t_c = p_c / q_c
        win = t_c > best_v
        best_v = jnp.where(win, t_c, best_v)
        best_c = jnp.where(win, c, best_c)
        return best_v, best_c

    best_v0 = jnp.full((_SUB, _CW), -np.float32(np.inf), jnp.float32)
    best_c0 = jnp.zeros((_SUB, _CW), jnp.int32)
    best_v, best_c = jax.lax.fori_loop(0, _NCH, pass_c, (best_v0, best_c0))

    mt = jnp.max(best_v)
    flat_m = sub_i * _LANES + best_c * _CW + lane_i
    idx_m = jnp.min(jnp.where(best_v == mt, flat_m, _BIG))

    idx = jnp.where(
        mt_t > mt, idx_t,
        jnp.where(mt_t == mt, jnp.minimum(idx_t, idx_m), idx_m))
    s_ref[0] = jnp.full((1, 128), idx, jnp.int32)


def kernel(logits):
    x3 = logits.reshape(_ROWS, _SUB, _LANES)
    row_spec = pl.BlockSpec((1, _SUB, _LANES), lambda i: (i, 0, 0))
    probs, samp = pl.pallas_call(
        _body,
        grid=(_ROWS,),
        in_specs=[row_spec],
        out_specs=[row_spec,
                   pl.BlockSpec((1, 1, 128), lambda i: (i, 0, 0))],
        out_shape=[jax.ShapeDtypeStruct((_ROWS, _SUB, _LANES), jnp.float32),
                   jax.ShapeDtypeStruct((_ROWS, 1, 128), jnp.int32)],
        scratch_shapes=[pltpu.VMEM((_SUB, _LANES), jnp.float32),
                        pltpu.VMEM((_SUB, _LANES), jnp.float32)],
    )(x3)
    return probs.reshape(_ROWS, _V), samp[:, 0, 0]


# speedup vs baseline: 1.1737x; 1.1737x over previous
import jax
import jax.numpy as jnp
import numpy as np
from jax.experimental import pallas as pl
from jax.experimental.pallas import tpu as pltpu

_ROWS = 32
_V = 1000000
_SUB = 8
_LANES = _V // _SUB
_CW = 128
_UNROLL = 4
_STEP = _CW * _UNROLL
_NIT = 244
_TAIL0 = _NIT * _STEP
_TAILW = _LANES - _TAIL0

_KS0 = np.uint32(0)
_KS1 = np.uint32(1)
_KS2 = np.uint32(0x1BD11BDB)
_ROT_A = (13, 15, 26, 6)
_ROT_B = (17, 29, 16, 24)
_BIG = np.int32(2 ** 30)


def _threefry_q(j):
    x0 = jnp.zeros_like(j) + _KS0
    x1 = j + _KS1
    sched = ((_ROT_A, _KS1, _KS2, 1), (_ROT_B, _KS2, _KS0, 2),
             (_ROT_A, _KS0, _KS1, 3), (_ROT_B, _KS1, _KS2, 4),
             (_ROT_A, _KS2, _KS0, 5))
    for rots, ka, kb, c in sched:
        for r in rots:
            x0 = x0 + x1
            x1 = (x1 << np.uint32(r)) | (x1 >> np.uint32(32 - r))
            x1 = x0 ^ x1
        x0 = x0 + ka
        x1 = x1 + kb + np.uint32(c)
    bits = x0 ^ x1
    fb = (bits >> np.uint32(9)) | np.uint32(0x3F800000)
    u = jnp.maximum(np.float32(0),
                    jax.lax.bitcast_convert_type(fb, jnp.float32)
                    - np.float32(1))
    return -jnp.log1p(-u)


def _iota2(shape):
    return (jax.lax.broadcasted_iota(jnp.int32, shape, 0),
            jax.lax.broadcasted_iota(jnp.int32, shape, 1))


def _body(x_ref, p_ref, s_ref, e_ref, q_ref):
    row = pl.program_id(0)
    m = jnp.max(x_ref[0])

    sub_i, lane_i = _iota2((_SUB, _CW))
    jbase = (sub_i * _LANES + lane_i + row * _V).astype(jnp.uint32)

    def pass_b(c, s_vec):
        base = c * _STEP
        for u in range(_UNROLL):
            off = base + u * _CW
            x_c = x_ref[0, :, pl.ds(off, _CW)]
            q_c = _threefry_q(jbase + off.astype(jnp.uint32))
            e_c = jnp.exp(x_c - m)
            e_ref[:, pl.ds(off, _CW)] = e_c
            q_ref[:, pl.ds(off, _CW)] = q_c
            s_vec = s_vec + e_c
        return s_vec

    s_vec = jax.lax.fori_loop(0, _NIT, pass_b,
                              jnp.zeros((_SUB, _CW), jnp.float32))

    sub_t, lane_t = _iota2((_SUB, _TAILW))
    jtail = (sub_t * _LANES + lane_t + (row * _V + _TAIL0)).astype(jnp.uint32)
    x_t = x_ref[0, :, pl.ds(_TAIL0, _TAILW)]
    q_t = _threefry_q(jtail)
    e_t = jnp.exp(x_t - m)
    s = jnp.sum(s_vec) + jnp.sum(e_t)
    p_t = e_t / s
    p_ref[0, :, pl.ds(_TAIL0, _TAILW)] = p_t
    t_t = p_t / q_t
    mt_t = jnp.max(t_t)
    flat_t = sub_t * _LANES + _TAIL0 + lane_t
    idx_t = jnp.min(jnp.where(t_t == mt_t, flat_t, _BIG))

    def pass_c(c, carry):
        best_v, best_c = carry
        base = c * _STEP
        for u in range(_UNROLL):
            off = base + u * _CW
            e_c = e_ref[:, pl.ds(off, _CW)]
            q_c = q_ref[:, pl.ds(off, _CW)]
            p_c = e_c / s
            p_ref[0, :, pl.ds(off, _CW)] = p_c
            t_c = p_c / q_c
            win = t_c > best_v
            best_v = jnp.where(win, t_c, best_v)
            best_c = jnp.where(win, c * _UNROLL + u, best_c)
        return best_v, best_c

    best_v0 = jnp.full((_SUB, _CW), -np.float32(np.inf), jnp.float32)
    best_c0 = jnp.zeros((_SUB, _CW), jnp.int32)
    best_v, best_c = jax.lax.fori_loop(0, _NIT, pass_c, (best_v0, best_c0))

    mt = jnp.max(best_v)
    flat_m = sub_i * _LANES + best_c * _CW + lane_i
    idx_m = jnp.min(jnp.where(best_v == mt, flat_m, _BIG))

    idx = jnp.where(
        mt_t > mt, idx_t,
        jnp.where(mt_t == mt, jnp.minimum(idx_t, idx_m), idx_m))
    s_ref[0] = jnp.full((1, 128), idx, jnp.int32)


def kernel(logits):
    x3 = logits.reshape(_ROWS, _SUB, _LANES)
    row_spec = pl.BlockSpec((1, _SUB, _LANES), lambda i: (i, 0, 0))
    probs, samp = pl.pallas_call(
        _body,
        grid=(_ROWS,),
        in_specs=[row_spec],
        out_specs=[row_spec,
                   pl.BlockSpec((1, 1, 128), lambda i: (i, 0, 0))],
        out_shape=[jax.ShapeDtypeStruct((_ROWS, _SUB, _LANES), jnp.float32),
                   jax.ShapeDtypeStruct((_ROWS, 1, 128), jnp.int32)],
        scratch_shapes=[pltpu.VMEM((_SUB, _LANES), jnp.float32),
                        pltpu.VMEM((_SUB, _LANES), jnp.float32)],
    )(x3)
    return probs.reshape(_ROWS, _V), samp[:, 0, 0]


# speedup vs baseline: 1.2353x; 1.0525x over previous
import jax
import jax.numpy as jnp
import numpy as np
from jax.experimental import pallas as pl
from jax.experimental.pallas import tpu as pltpu

_ROWS = 32
_V = 1000000
_SUB = 8
_LANES = _V // _SUB
_CW = 128
_UNROLL = 8
_STEP = _CW * _UNROLL
_NIT = 122
_TAIL0 = _NIT * _STEP
_TAILW = _LANES - _TAIL0

_KS0 = np.uint32(0)
_KS1 = np.uint32(1)
_KS2 = np.uint32(0x1BD11BDB)
_ROT_A = (13, 15, 26, 6)
_ROT_B = (17, 29, 16, 24)
_BIG = np.int32(2 ** 30)


def _threefry_q(j):
    x0 = jnp.zeros_like(j) + _KS0
    x1 = j + _KS1
    sched = ((_ROT_A, _KS1, _KS2, 1), (_ROT_B, _KS2, _KS0, 2),
             (_ROT_A, _KS0, _KS1, 3), (_ROT_B, _KS1, _KS2, 4),
             (_ROT_A, _KS2, _KS0, 5))
    for rots, ka, kb, c in sched:
        for r in rots:
            x0 = x0 + x1
            x1 = (x1 << np.uint32(r)) | (x1 >> np.uint32(32 - r))
            x1 = x0 ^ x1
        x0 = x0 + ka
        x1 = x1 + kb + np.uint32(c)
    bits = x0 ^ x1
    fb = (bits >> np.uint32(9)) | np.uint32(0x3F800000)
    u = jnp.maximum(np.float32(0),
                    jax.lax.bitcast_convert_type(fb, jnp.float32)
                    - np.float32(1))
    return -jnp.log1p(-u)


def _iota2(shape):
    return (jax.lax.broadcasted_iota(jnp.int32, shape, 0),
            jax.lax.broadcasted_iota(jnp.int32, shape, 1))


def _body(x_ref, p_ref, s_ref, e_ref):
    row = pl.program_id(0)
    m = jnp.max(x_ref[0])

    sub_i, lane_i = _iota2((_SUB, _CW))
    jbase = (sub_i * _LANES + lane_i + row * _V).astype(jnp.uint32)

    def pass_b(i, carry):
        s_vec, eb, qb, cb = carry
        base = i * _STEP
        for u in range(_UNROLL):
            off = base + u * _CW
            x_c = x_ref[0, :, pl.ds(off, _CW)]
            q_c = _threefry_q(jbase + off.astype(jnp.uint32))
            e_c = jnp.exp(x_c - m)
            e_ref[:, pl.ds(off, _CW)] = e_c
            s_vec = s_vec + e_c
            win = e_c * qb > eb * q_c
            eb = jnp.where(win, e_c, eb)
            qb = jnp.where(win, q_c, qb)
            cb = jnp.where(win, i * _UNROLL + u, cb)
        return s_vec, eb, qb, cb

    carry0 = (jnp.zeros((_SUB, _CW), jnp.float32),
              jnp.zeros((_SUB, _CW), jnp.float32),
              jnp.ones((_SUB, _CW), jnp.float32),
              jnp.zeros((_SUB, _CW), jnp.int32))
    s_vec, eb, qb, cb = jax.lax.fori_loop(0, _NIT, pass_b, carry0)

    sub_t, lane_t = _iota2((_SUB, _TAILW))
    jtail = (sub_t * _LANES + lane_t + (row * _V + _TAIL0)).astype(jnp.uint32)
    x_t = x_ref[0, :, pl.ds(_TAIL0, _TAILW)]
    q_t = _threefry_q(jtail)
    e_t = jnp.exp(x_t - m)
    s = jnp.sum(s_vec) + jnp.sum(e_t)
    rs = np.float32(1) / s

    p_t = e_t * rs
    p_ref[0, :, pl.ds(_TAIL0, _TAILW)] = p_t
    t_t = e_t / q_t
    mt_t = jnp.max(t_t)
    flat_t = sub_t * _LANES + _TAIL0 + lane_t
    idx_t = jnp.min(jnp.where(t_t == mt_t, flat_t, _BIG))

    def pass_c(i, _):
        base = i * _STEP
        for u in range(_UNROLL):
            off = base + u * _CW
            p_ref[0, :, pl.ds(off, _CW)] = e_ref[:, pl.ds(off, _CW)] * rs
        return 0

    jax.lax.fori_loop(0, _NIT, pass_c, 0)

    t_m = eb / qb
    mt = jnp.max(t_m)
    flat_m = sub_i * _LANES + cb * _CW + lane_i
    idx_m = jnp.min(jnp.where(t_m == mt, flat_m, _BIG))

    idx = jnp.where(
        mt_t > mt, idx_t,
        jnp.where(mt_t == mt, jnp.minimum(idx_t, idx_m), idx_m))
    s_ref[0] = jnp.full((1, 128), idx, jnp.int32)


def kernel(logits):
    x3 = logits.reshape(_ROWS, _SUB, _LANES)
    row_spec = pl.BlockSpec((1, _SUB, _LANES), lambda i: (i, 0, 0))
    probs, samp = pl.pallas_call(
        _body,
        grid=(_ROWS,),
        in_specs=[row_spec],
        out_specs=[row_spec,
                   pl.BlockSpec((1, 1, 128), lambda i: (i, 0, 0))],
        out_shape=[jax.ShapeDtypeStruct((_ROWS, _SUB, _LANES), jnp.float32),
                   jax.ShapeDtypeStruct((_ROWS, 1, 128), jnp.int32)],
        scratch_shapes=[pltpu.VMEM((_SUB, _LANES), jnp.float32)],
    )(x3)
    return probs.reshape(_ROWS, _V), samp[:, 0, 0]


# speedup vs baseline: 1.6131x; 1.3058x over previous
import jax
import jax.numpy as jnp
import numpy as np
from jax.experimental import pallas as pl
from jax.experimental.pallas import tpu as pltpu

_ROWS = 32
_V = 1000000
_SUB = 8
_NGRP = _ROWS // _SUB
_CW = 128
_UNROLL = 6
_STEP = _CW * _UNROLL
_NIT = 1302
_TAIL0 = _NIT * _STEP
_TAILW = _V - _TAIL0

_KS0 = np.uint32(0)
_KS1 = np.uint32(1)
_KS2 = np.uint32(0x1BD11BDB)
_ROT_A = (13, 15, 26, 6)
_ROT_B = (17, 29, 16, 24)
_BIG = np.int32(2 ** 30)


def _threefry_q(j):
    x0 = jnp.zeros_like(j) + _KS0
    x1 = j + _KS1
    sched = ((_ROT_A, _KS1, _KS2, 1), (_ROT_B, _KS2, _KS0, 2),
             (_ROT_A, _KS0, _KS1, 3), (_ROT_B, _KS1, _KS2, 4),
             (_ROT_A, _KS2, _KS0, 5))
    for rots, ka, kb, c in sched:
        for r in rots:
            x0 = x0 + x1
            x1 = (x1 << np.uint32(r)) | (x1 >> np.uint32(32 - r))
            x1 = x0 ^ x1
        x0 = x0 + ka
        x1 = x1 + kb + np.uint32(c)
    bits = x0 ^ x1
    fb = (bits >> np.uint32(9)) | np.uint32(0x3F800000)
    u = jnp.maximum(np.float32(0),
                    jax.lax.bitcast_convert_type(fb, jnp.float32)
                    - np.float32(1))
    return -jnp.log1p(-u)


def _iota2(shape):
    return (jax.lax.broadcasted_iota(jnp.int32, shape, 0),
            jax.lax.broadcasted_iota(jnp.int32, shape, 1))


def _body(x_hbm, p_hbm, s_ref, xbuf, ebuf, sem_x, sem_p):
    g = pl.program_id(0)
    rows = pl.ds(g * _SUB, _SUB)
    pltpu.make_async_copy(x_hbm.at[rows], xbuf, sem_x).start()
    pltpu.make_async_copy(x_hbm.at[rows], xbuf, sem_x).wait()

    m_vec = jnp.max(xbuf[...], axis=1, keepdims=True)

    sub_i, lane_i = _iota2((_SUB, _CW))
    jbase = (sub_i * _V + lane_i + g * (_SUB * _V)).astype(jnp.uint32)

    def pass_b(i, carry):
        s_vec, eb, qb, cb = carry
        base = i * _STEP
        for u in range(_UNROLL):
            off = base + u * _CW
            x_c = xbuf[:, pl.ds(off, _CW)]
            q_c = _threefry_q(jbase + off.astype(jnp.uint32))
            e_c = jnp.exp(x_c - m_vec)
            ebuf[:, pl.ds(off, _CW)] = e_c
            s_vec = s_vec + e_c
            win = e_c * qb > eb * q_c
            eb = jnp.where(win, e_c, eb)
            qb = jnp.where(win, q_c, qb)
            cb = jnp.where(win, i * _UNROLL + u, cb)
        return s_vec, eb, qb, cb

    carry0 = (jnp.zeros((_SUB, _CW), jnp.float32),
              jnp.zeros((_SUB, _CW), jnp.float32),
              jnp.ones((_SUB, _CW), jnp.float32),
              jnp.zeros((_SUB, _CW), jnp.int32))
    s_vec, eb, qb, cb = jax.lax.fori_loop(0, _NIT, pass_b, carry0)

    sub_t, lane_t = _iota2((_SUB, _TAILW))
    jtail = (sub_t * _V + lane_t + (g * (_SUB * _V) + _TAIL0)
             ).astype(jnp.uint32)
    x_t = xbuf[:, pl.ds(_TAIL0, _TAILW)]
    q_t = _threefry_q(jtail)
    e_t = jnp.exp(x_t - m_vec)
    s_row = (jnp.sum(s_vec, axis=1, keepdims=True)
             + jnp.sum(e_t, axis=1, keepdims=True))
    rs = np.float32(1) / s_row

    xbuf[:, pl.ds(_TAIL0, _TAILW)] = e_t * rs
    t_t = e_t / q_t
    mt_t = jnp.max(t_t, axis=1, keepdims=True)
    flat_t = _TAIL0 + lane_t
    idx_t = jnp.min(jnp.where(t_t == mt_t, flat_t, _BIG),
                    axis=1, keepdims=True)

    def pass_c(i, _):
        base = i * _STEP
        for u in range(_UNROLL):
            off = base + u * _CW
            xbuf[:, pl.ds(off, _CW)] = ebuf[:, pl.ds(off, _CW)] * rs
        return 0

    jax.lax.fori_loop(0, _NIT, pass_c, 0)

    pltpu.make_async_copy(xbuf, p_hbm.at[rows], sem_p).start()

    t_m = eb / qb
    mt = jnp.max(t_m, axis=1, keepdims=True)
    flat_m = cb * _CW + lane_i
    idx_m = jnp.min(jnp.where(t_m == mt, flat_m, _BIG),
                    axis=1, keepdims=True)

    idx = jnp.where(
        mt_t > mt, idx_t,
        jnp.where(mt_t == mt, jnp.minimum(idx_t, idx_m), idx_m))
    s_ref[...] = jnp.broadcast_to(idx, (_SUB, _CW))

    pltpu.make_async_copy(xbuf, p_hbm.at[rows], sem_p).wait()


def kernel(logits):
    probs, samp = pl.pallas_call(
        _body,
        grid=(_NGRP,),
        in_specs=[pl.BlockSpec(memory_space=pl.ANY)],
        out_specs=[pl.BlockSpec(memory_space=pl.ANY),
                   pl.BlockSpec((_SUB, _CW), lambda i: (i, 0))],
        out_shape=[jax.ShapeDtypeStruct((_ROWS, _V), jnp.float32),
                   jax.ShapeDtypeStruct((_ROWS, _CW), jnp.int32)],
        scratch_shapes=[pltpu.VMEM((_SUB, _V), jnp.float32),
                        pltpu.VMEM((_SUB, _V), jnp.float32),
                        pltpu.SemaphoreType.DMA,
                        pltpu.SemaphoreType.DMA],
        compiler_params=pltpu.CompilerParams(
            vmem_limit_bytes=100 * 1024 * 1024),
    )(logits)
    return probs, samp[:, 0]


# speedup vs baseline: 1.7323x; 1.0739x over previous
import jax
import jax.numpy as jnp
import numpy as np
from jax.experimental import pallas as pl
from jax.experimental.pallas import tpu as pltpu

_ROWS = 32
_V = 1000000
_SUB = 8
_NGRP = _ROWS // _SUB
_CW = 128
_UNROLL = 12
_STEP = _CW * _UNROLL
_NIT = 651
_TAIL0 = _NIT * _STEP
_TAILW = _V - _TAIL0

_KS0 = np.uint32(0)
_KS1 = np.uint32(1)
_KS2 = np.uint32(0x1BD11BDB)
_ROT_A = (13, 15, 26, 6)
_ROT_B = (17, 29, 16, 24)
_BIG = np.int32(2 ** 30)


def _threefry_q(j):
    x0 = jnp.zeros_like(j) + _KS0
    x1 = j + _KS1
    sched = ((_ROT_A, _KS1, _KS2, 1), (_ROT_B, _KS2, _KS0, 2),
             (_ROT_A, _KS0, _KS1, 3), (_ROT_B, _KS1, _KS2, 4),
             (_ROT_A, _KS2, _KS0, 5))
    for rots, ka, kb, c in sched:
        for r in rots:
            x0 = x0 + x1
            x1 = (x1 << np.uint32(r)) | (x1 >> np.uint32(32 - r))
            x1 = x0 ^ x1
        x0 = x0 + ka
        x1 = x1 + kb + np.uint32(c)
    bits = x0 ^ x1
    fb = (bits >> np.uint32(9)) | np.uint32(0x3F800000)
    u = jnp.maximum(np.float32(0),
                    jax.lax.bitcast_convert_type(fb, jnp.float32)
                    - np.float32(1))
    return -jnp.log1p(-u)


def _iota2(shape):
    return (jax.lax.broadcasted_iota(jnp.int32, shape, 0),
            jax.lax.broadcasted_iota(jnp.int32, shape, 1))


def _body(x_hbm, p_hbm, s_ref, xbuf, ebuf, sem_x, sem_p):
    g = pl.program_id(0)
    rows = pl.ds(g * _SUB, _SUB)

    pltpu.make_async_copy(x_hbm.at[rows], xbuf, sem_x).start()
    pltpu.make_async_copy(x_hbm.at[rows], xbuf, sem_x).wait()

    def pass_a(i, mv):
        base = i * _STEP
        for u in range(_UNROLL):
            mv = jnp.maximum(mv, xbuf[:, pl.ds(base + u * _CW, _CW)])
        return mv

    mv0 = jnp.full((_SUB, _CW), -np.float32(np.inf), jnp.float32)
    mv = jax.lax.fori_loop(0, _NIT, pass_a, mv0)
    x_t = xbuf[:, pl.ds(_TAIL0, _TAILW)]
    m_vec = jnp.maximum(jnp.max(mv, axis=1, keepdims=True),
                        jnp.max(x_t, axis=1, keepdims=True))

    @pl.when(g > 0)
    def _():
        prev = pl.ds((g - 1) * _SUB, _SUB)
        pltpu.make_async_copy(ebuf, p_hbm.at[prev], sem_p).wait()

    sub_i, lane_i = _iota2((_SUB, _CW))
    jbase = (sub_i * _V + lane_i + g * (_SUB * _V)).astype(jnp.uint32)

    def pass_b(i, carry):
        s_vec, eb, qb, cb = carry
        base = i * _STEP
        for u in range(_UNROLL):
            off = base + u * _CW
            x_c = xbuf[:, pl.ds(off, _CW)]
            q_c = _threefry_q(jbase + off.astype(jnp.uint32))
            e_c = jnp.exp(x_c - m_vec)
            ebuf[:, pl.ds(off, _CW)] = e_c
            s_vec = s_vec + e_c
            win = e_c * qb > eb * q_c
            eb = jnp.where(win, e_c, eb)
            qb = jnp.where(win, q_c, qb)
            cb = jnp.where(win, i * _UNROLL + u, cb)
        return s_vec, eb, qb, cb

    carry0 = (jnp.zeros((_SUB, _CW), jnp.float32),
              jnp.zeros((_SUB, _CW), jnp.float32),
              jnp.ones((_SUB, _CW), jnp.float32),
              jnp.zeros((_SUB, _CW), jnp.int32))
    s_vec, eb, qb, cb = jax.lax.fori_loop(0, _NIT, pass_b, carry0)

    sub_t, lane_t = _iota2((_SUB, _TAILW))
    jtail = (sub_t * _V + lane_t + (g * (_SUB * _V) + _TAIL0)
             ).astype(jnp.uint32)
    q_t = _threefry_q(jtail)
    e_t = jnp.exp(x_t - m_vec)
    s_row = (jnp.sum(s_vec, axis=1, keepdims=True)
             + jnp.sum(e_t, axis=1, keepdims=True))
    rs = np.float32(1) / s_row

    ebuf[:, pl.ds(_TAIL0, _TAILW)] = e_t * rs
    t_t = e_t / q_t
    mt_t = jnp.max(t_t, axis=1, keepdims=True)
    flat_t = _TAIL0 + lane_t
    idx_t = jnp.min(jnp.where(t_t == mt_t, flat_t, _BIG),
                    axis=1, keepdims=True)

    def pass_c(i, _):
        base = i * _STEP
        for u in range(_UNROLL):
            off = base + u * _CW
            ebuf[:, pl.ds(off, _CW)] = ebuf[:, pl.ds(off, _CW)] * rs
        return 0

    jax.lax.fori_loop(0, _NIT, pass_c, 0)

    pltpu.make_async_copy(ebuf, p_hbm.at[rows], sem_p).start()

    t_m = eb / qb
    mt = jnp.max(t_m, axis=1, keepdims=True)
    flat_m = cb * _CW + lane_i
    idx_m = jnp.min(jnp.where(t_m == mt, flat_m, _BIG),
                    axis=1, keepdims=True)

    idx = jnp.where(
        mt_t > mt, idx_t,
        jnp.where(mt_t == mt, jnp.minimum(idx_t, idx_m), idx_m))
    s_ref[...] = jnp.broadcast_to(idx, (_SUB, _CW))

    @pl.when(g == _NGRP - 1)
    def _():
        pltpu.make_async_copy(ebuf, p_hbm.at[rows], sem_p).wait()


def kernel(logits):
    probs, samp = pl.pallas_call(
        _body,
        grid=(_NGRP,),
        in_specs=[pl.BlockSpec(memory_space=pl.ANY)],
        out_specs=[pl.BlockSpec(memory_space=pl.ANY),
                   pl.BlockSpec((_SUB, _CW), lambda i: (i, 0))],
        out_shape=[jax.ShapeDtypeStruct((_ROWS, _V), jnp.float32),
                   jax.ShapeDtypeStruct((_ROWS, _CW), jnp.int32)],
        scratch_shapes=[pltpu.VMEM((_SUB, _V), jnp.float32),
                        pltpu.VMEM((_SUB, _V), jnp.float32),
                        pltpu.SemaphoreType.DMA,
                        pltpu.SemaphoreType.DMA],
        compiler_params=pltpu.CompilerParams(
            vmem_limit_bytes=100 * 1024 * 1024),
    )(logits)
    return probs, samp[:, 0]


# speedup vs baseline: 1.9945x; 1.1514x over previous
import jax
import jax.numpy as jnp
import numpy as np
from jax.experimental import pallas as pl
from jax.experimental.pallas import tpu as pltpu

_ROWS = 32
_V = 1000000
_SUB = 8
_NGRP = _ROWS // _SUB
_CW = 128
_UNROLL = 12
_STEP = _CW * _UNROLL
_NIT = 651
_TAIL0 = _NIT * _STEP
_TAILW = _V - _TAIL0

_KS0 = np.uint32(0)
_KS1 = np.uint32(1)
_KS2 = np.uint32(0x1BD11BDB)
_ROT_A = (13, 15, 26, 6)
_ROT_B = (17, 29, 16, 24)
_BIG = np.int32(2 ** 30)


def _threefry_q(j):
    x0 = jnp.zeros_like(j) + _KS0
    x1 = j + _KS1
    sched = ((_ROT_A, _KS1, _KS2, 1), (_ROT_B, _KS2, _KS0, 2),
             (_ROT_A, _KS0, _KS1, 3), (_ROT_B, _KS1, _KS2, 4),
             (_ROT_A, _KS2, _KS0, 5))
    for rots, ka, kb, c in sched:
        for r in rots:
            x0 = x0 + x1
            x1 = (x1 << np.uint32(r)) | (x1 >> np.uint32(32 - r))
            x1 = x0 ^ x1
        x0 = x0 + ka
        x1 = x1 + kb + np.uint32(c)
    bits = x0 ^ x1
    fb = (bits >> np.uint32(9)) | np.uint32(0x3F800000)
    u = jnp.maximum(np.float32(0),
                    jax.lax.bitcast_convert_type(fb, jnp.float32)
                    - np.float32(1))
    return -jnp.log1p(-u)


def _iota2(shape):
    return (jax.lax.broadcasted_iota(jnp.int32, shape, 0),
            jax.lax.broadcasted_iota(jnp.int32, shape, 1))


def _body(x_hbm, p_hbm, s_ref, xbuf, ebuf, sem_x, sem_p):
    g = pl.program_id(0)
    rows = pl.ds(g * _SUB, _SUB)

    pltpu.make_async_copy(x_hbm.at[rows], xbuf, sem_x).start()
    pltpu.make_async_copy(x_hbm.at[rows], xbuf, sem_x).wait()

    def pass_a(i, mv):
        x_slab = xbuf[:, pl.ds(i * _STEP, _STEP)]
        return jnp.maximum(mv, x_slab)

    mv0 = jnp.full((_SUB, _STEP), -np.float32(np.inf), jnp.float32)
    mv = jax.lax.fori_loop(0, _NIT, pass_a, mv0)
    mvr = mv[:, :_CW]
    for u in range(1, _UNROLL):
        mvr = jnp.maximum(mvr, mv[:, u * _CW:(u + 1) * _CW])
    mv = mvr
    x_t = xbuf[:, pl.ds(_TAIL0, _TAILW)]
    m_vec = jnp.maximum(jnp.max(mv, axis=1, keepdims=True),
                        jnp.max(x_t, axis=1, keepdims=True))

    @pl.when(g > 0)
    def _():
        prev = pl.ds((g - 1) * _SUB, _SUB)
        pltpu.make_async_copy(ebuf, p_hbm.at[prev], sem_p).wait()

    sub_i, lane_i = _iota2((_SUB, _CW))
    jbase = (sub_i * _V + lane_i + g * (_SUB * _V)).astype(jnp.uint32)

    def pass_b(i, carry):
        s_vec, eb, qb, cb = carry
        base = i * _STEP
        x_slab = xbuf[:, pl.ds(base, _STEP)]
        e_parts = []
        for u in range(_UNROLL):
            x_c = x_slab[:, u * _CW:(u + 1) * _CW]
            q_c = _threefry_q(jbase + base.astype(jnp.uint32)
                              + np.uint32(u * _CW))
            e_c = jnp.exp(x_c - m_vec)
            e_parts.append(e_c)
            s_vec = s_vec + e_c
            win = e_c * qb > eb * q_c
            eb = jnp.where(win, e_c, eb)
            qb = jnp.where(win, q_c, qb)
            cb = jnp.where(win, i * _UNROLL + u, cb)
        ebuf[:, pl.ds(base, _STEP)] = jnp.concatenate(e_parts, axis=1)
        return s_vec, eb, qb, cb

    carry0 = (jnp.zeros((_SUB, _CW), jnp.float32),
              jnp.zeros((_SUB, _CW), jnp.float32),
              jnp.ones((_SUB, _CW), jnp.float32),
              jnp.zeros((_SUB, _CW), jnp.int32))
    s_vec, eb, qb, cb = jax.lax.fori_loop(0, _NIT, pass_b, carry0)

    sub_t, lane_t = _iota2((_SUB, _TAILW))
    jtail = (sub_t * _V + lane_t + (g * (_SUB * _V) + _TAIL0)
             ).astype(jnp.uint32)
    q_t = _threefry_q(jtail)
    e_t = jnp.exp(x_t - m_vec)
    s_row = (jnp.sum(s_vec, axis=1, keepdims=True)
             + jnp.sum(e_t, axis=1, keepdims=True))
    rs = np.float32(1) / s_row

    ebuf[:, pl.ds(_TAIL0, _TAILW)] = e_t * rs
    t_t = e_t / q_t
    mt_t = jnp.max(t_t, axis=1, keepdims=True)
    flat_t = _TAIL0 + lane_t
    idx_t = jnp.min(jnp.where(t_t == mt_t, flat_t, _BIG),
                    axis=1, keepdims=True)

    def pass_c(i, _):
        sl = pl.ds(i * _STEP, _STEP)
        ebuf[:, sl] = ebuf[:, sl] * rs
        return 0

    jax.lax.fori_loop(0, _NIT, pass_c, 0)

    pltpu.make_async_copy(ebuf, p_hbm.at[rows], sem_p).start()

    t_m = eb / qb
    mt = jnp.max(t_m, axis=1, keepdims=True)
    flat_m = cb * _CW + lane_i
    idx_m = jnp.min(jnp.where(t_m == mt, flat_m, _BIG),
                    axis=1, keepdims=True)

    idx = jnp.where(
        mt_t > mt, idx_t,
        jnp.where(mt_t == mt, jnp.minimum(idx_t, idx_m), idx_m))
    s_ref[...] = jnp.broadcast_to(idx, (_SUB, _CW))

    @pl.when(g == _NGRP - 1)
    def _():
        pltpu.make_async_copy(ebuf, p_hbm.at[rows], sem_p).wait()


def kernel(logits):
    probs, samp = pl.pallas_call(
        _body,
        grid=(_NGRP,),
        in_specs=[pl.BlockSpec(memory_space=pl.ANY)],
        out_specs=[pl.BlockSpec(memory_space=pl.ANY),
                   pl.BlockSpec((_SUB, _CW), lambda i: (i, 0))],
        out_shape=[jax.ShapeDtypeStruct((_ROWS, _V), jnp.float32),
                   jax.ShapeDtypeStruct((_ROWS, _CW), jnp.int32)],
        scratch_shapes=[pltpu.VMEM((_SUB, _V), jnp.float32),
                        pltpu.VMEM((_SUB, _V), jnp.float32),
                        pltpu.SemaphoreType.DMA,
                        pltpu.SemaphoreType.DMA],
        compiler_params=pltpu.CompilerParams(
            vmem_limit_bytes=100 * 1024 * 1024),
    )(logits)
    return probs, samp[:, 0]


# speedup vs baseline: 2.0938x; 1.0498x over previous
import jax
import jax.numpy as jnp
import numpy as np
from jax.experimental import pallas as pl
from jax.experimental.pallas import tpu as pltpu

_ROWS = 32
_V = 1000000
_SUB = 8
_NGRP = _ROWS // _SUB
_CW = 128
_UNROLL = 12
_STEP = _CW * _UNROLL
_NIT = 651
_TAIL0 = _NIT * _STEP
_TAILW = _V - _TAIL0

_KS0 = np.uint32(0)
_KS1 = np.uint32(1)
_KS2 = np.uint32(0x1BD11BDB)
_ROT_A = (13, 15, 26, 6)
_ROT_B = (17, 29, 16, 24)
_BIG = np.int32(2 ** 30)


def _threefry_q(j):
    x0 = jnp.zeros_like(j) + _KS0
    x1 = j + _KS1
    sched = ((_ROT_A, _KS1, _KS2, 1), (_ROT_B, _KS2, _KS0, 2),
             (_ROT_A, _KS0, _KS1, 3), (_ROT_B, _KS1, _KS2, 4),
             (_ROT_A, _KS2, _KS0, 5))
    for rots, ka, kb, c in sched:
        for r in rots:
            x0 = x0 + x1
            x1 = (x1 << np.uint32(r)) | (x1 >> np.uint32(32 - r))
            x1 = x0 ^ x1
        x0 = x0 + ka
        x1 = x1 + kb + np.uint32(c)
    bits = x0 ^ x1
    fb = (bits >> np.uint32(9)) | np.uint32(0x3F800000)
    u = jnp.maximum(np.float32(0),
                    jax.lax.bitcast_convert_type(fb, jnp.float32)
                    - np.float32(1))
    return -jnp.log1p(-u)


def _iota2(shape):
    return (jax.lax.broadcasted_iota(jnp.int32, shape, 0),
            jax.lax.broadcasted_iota(jnp.int32, shape, 1))


def _body(x_hbm, p_hbm, s_ref, xbuf, ebuf, sem_x, sem_p):
    g = pl.program_id(0)
    rows = pl.ds(g * _SUB, _SUB)

    @pl.when(g == 0)
    def _():
        pltpu.make_async_copy(x_hbm.at[rows], xbuf, sem_x).start()

    pltpu.make_async_copy(x_hbm.at[rows], xbuf, sem_x).wait()

    def pass_a(i, mv):
        x_slab = xbuf[:, pl.ds(i * _STEP, _STEP)]
        return jnp.maximum(mv, x_slab)

    mv0 = jnp.full((_SUB, _STEP), -np.float32(np.inf), jnp.float32)
    mv = jax.lax.fori_loop(0, _NIT, pass_a, mv0)
    mvr = mv[:, :_CW]
    for u in range(1, _UNROLL):
        mvr = jnp.maximum(mvr, mv[:, u * _CW:(u + 1) * _CW])
    mv = mvr
    x_t = xbuf[:, pl.ds(_TAIL0, _TAILW)]
    m_vec = jnp.maximum(jnp.max(mv, axis=1, keepdims=True),
                        jnp.max(x_t, axis=1, keepdims=True))

    @pl.when(g > 0)
    def _():
        prev = pl.ds((g - 1) * _SUB, _SUB)
        pltpu.make_async_copy(ebuf, p_hbm.at[prev], sem_p).wait()

    sub_i, lane_i = _iota2((_SUB, _CW))
    jbase = (sub_i * _V + lane_i + g * (_SUB * _V)).astype(jnp.uint32)

    def pass_b(i, carry):
        s_vec, eb, qb, cb = carry
        base = i * _STEP
        x_slab = xbuf[:, pl.ds(base, _STEP)]
        e_parts = []
        for u in range(_UNROLL):
            x_c = x_slab[:, u * _CW:(u + 1) * _CW]
            q_c = _threefry_q(jbase + base.astype(jnp.uint32)
                              + np.uint32(u * _CW))
            e_c = jnp.exp(x_c - m_vec)
            e_parts.append(e_c)
            s_vec = s_vec + e_c
            win = e_c * qb > eb * q_c
            eb = jnp.where(win, e_c, eb)
            qb = jnp.where(win, q_c, qb)
            cb = jnp.where(win, i * _UNROLL + u, cb)
        ebuf[:, pl.ds(base, _STEP)] = jnp.concatenate(e_parts, axis=1)
        return s_vec, eb, qb, cb

    carry0 = (jnp.zeros((_SUB, _CW), jnp.float32),
              jnp.zeros((_SUB, _CW), jnp.float32),
              jnp.ones((_SUB, _CW), jnp.float32),
              jnp.zeros((_SUB, _CW), jnp.int32))
    s_vec, eb, qb, cb = jax.lax.fori_loop(0, _NIT, pass_b, carry0)

    sub_t, lane_t = _iota2((_SUB, _TAILW))
    jtail = (sub_t * _V + lane_t + (g * (_SUB * _V) + _TAIL0)
             ).astype(jnp.uint32)
    q_t = _threefry_q(jtail)
    e_t = jnp.exp(x_t - m_vec)
    s_row = (jnp.sum(s_vec, axis=1, keepdims=True)
             + jnp.sum(e_t, axis=1, keepdims=True))
    rs = np.float32(1) / s_row

    ebuf[:, pl.ds(_TAIL0, _TAILW)] = e_t * rs
    t_t = e_t / q_t
    mt_t = jnp.max(t_t, axis=1, keepdims=True)
    flat_t = _TAIL0 + lane_t
    idx_t = jnp.min(jnp.where(t_t == mt_t, flat_t, _BIG),
                    axis=1, keepdims=True)

    @pl.when(g + 1 < _NGRP)
    def _():
        nxt = pl.ds((g + 1) * _SUB, _SUB)
        pltpu.make_async_copy(x_hbm.at[nxt], xbuf, sem_x).start()

    def pass_c(i, _):
        sl = pl.ds(i * _STEP, _STEP)
        ebuf[:, sl] = ebuf[:, sl] * rs
        return 0

    jax.lax.fori_loop(0, _NIT, pass_c, 0)

    pltpu.make_async_copy(ebuf, p_hbm.at[rows], sem_p).start()

    t_m = eb / qb
    mt = jnp.max(t_m, axis=1, keepdims=True)
    flat_m = cb * _CW + lane_i
    idx_m = jnp.min(jnp.where(t_m == mt, flat_m, _BIG),
                    axis=1, keepdims=True)

    idx = jnp.where(
        mt_t > mt, idx_t,
        jnp.where(mt_t == mt, jnp.minimum(idx_t, idx_m), idx_m))
    s_ref[...] = jnp.broadcast_to(idx, (_SUB, _CW))

    @pl.when(g == _NGRP - 1)
    def _():
        pltpu.make_async_copy(ebuf, p_hbm.at[rows], sem_p).wait()


def kernel(logits):
    probs, samp = pl.pallas_call(
        _body,
        grid=(_NGRP,),
        in_specs=[pl.BlockSpec(memory_space=pl.ANY)],
        out_specs=[pl.BlockSpec(memory_space=pl.ANY),
                   pl.BlockSpec((_SUB, _CW), lambda i: (i, 0))],
        out_shape=[jax.ShapeDtypeStruct((_ROWS, _V), jnp.float32),
                   jax.ShapeDtypeStruct((_ROWS, _CW), jnp.int32)],
        scratch_shapes=[pltpu.VMEM((_SUB, _V), jnp.float32),
                        pltpu.VMEM((_SUB, _V), jnp.float32),
                        pltpu.SemaphoreType.DMA,
                        pltpu.SemaphoreType.DMA],
        compiler_params=pltpu.CompilerParams(
            vmem_limit_bytes=100 * 1024 * 1024),
    )(logits)
    return probs, samp[:, 0]
